# baseline (speedup 1.0000x reference)
import functools

import jax
import jax.numpy as jnp
from jax import lax
from jax.experimental import pallas as pl
from jax.experimental.pallas import tpu as pltpu
from jax.experimental.pallas import tpu_sc as plsc

_D = 64
_CHUNK = 128
_H = 50
_BT = 128

_INFO = plsc.get_sparse_core_info()
_NC = _INFO.num_cores
_NS = _INFO.num_subcores
_NW = _NC * _NS
_BT_PER_W = _BT // _NW
_N_CHUNKS = _H * _BT_PER_W


def _emb_body(idx_hbm, table_hbm, out_hbm, idx_v, rows0, rows1, patch0,
              patch1, gsem, psem):
    wid = lax.axis_index("s") * _NC + lax.axis_index("c")
    pltpu.sync_copy(idx_hbm.at[wid], idx_v)

    rows = (rows0, rows1)
    patch = (patch0, patch1)

    scat = [(lax.iota(jnp.int32, 16) + 16 * g) * 128 for g in range(4)]

    def issue_gather(j, b):
        pltpu.async_copy(table_hbm.at[idx_v.at[j]], rows[b], gsem.at[b])

    def wait_gather(j, b):
        pltpu.make_async_copy(
            table_hbm.at[idx_v.at[j]], rows[b], gsem.at[b]).wait()

    def transpose_chunk(b):
        for bl in range(_CHUNK):
            for g in range(4):
                vals = rows[b][bl, pl.ds(16 * g, 16)]
                plsc.store_scatter(patch[b], [scat[g] + bl], vals)

    def out_tile(j, dt):
        h = j // _BT_PER_W
        bt = wid * _BT_PER_W + (j % _BT_PER_W)
        return out_hbm.at[h, dt, bt]

    def issue_writes(j, b):
        for dt in range(8):
            pltpu.async_copy(
                patch[b].at[pl.ds(1024 * dt, 1024)], out_tile(j, dt),
                psem.at[b])

    def wait_writes(j, b):
        for dt in range(8):
            pltpu.make_async_copy(
                patch[b].at[pl.ds(1024 * dt, 1024)], out_tile(j, dt),
                psem.at[b]).wait()

    issue_gather(0, 0)

    def pair(p, carry):
        for s in range(2):
            j = 2 * p + s
            wait_gather(j, s)

            @pl.when(j + 1 < _N_CHUNKS)
            def _():
                issue_gather(j + 1, 1 - s)

            @pl.when(j >= 2)
            def _():
                wait_writes(j - 2, s)

            transpose_chunk(s)
            issue_writes(j, s)
        return carry

    lax.fori_loop(0, _N_CHUNKS // 2, pair, 0)

    wait_writes(_N_CHUNKS - 2, 0)
    wait_writes(_N_CHUNKS - 1, 1)


@jax.jit
def _emb_call(idx, weight):
    mesh = plsc.VectorSubcoreMesh(core_axis_name="c", subcore_axis_name="s")
    run = pl.kernel(
        _emb_body,
        out_type=jax.ShapeDtypeStruct((_H, 8, _BT, 8 * _CHUNK), jnp.float32),
        mesh=mesh,
        scratch_types=[
            pltpu.VMEM((_N_CHUNKS, _CHUNK), jnp.int32),
            pltpu.VMEM((_CHUNK, _D), jnp.float32),
            pltpu.VMEM((_CHUNK, _D), jnp.float32),
            pltpu.VMEM((_D * _CHUNK,), jnp.float32),
            pltpu.VMEM((_D * _CHUNK,), jnp.float32),
            pltpu.SemaphoreType.DMA((2,)),
            pltpu.SemaphoreType.DMA((2,)),
        ],
        compiler_params=pltpu.CompilerParams(
            use_tc_tiling_on_sc=False, needs_layout_passes=False),
    )
    return run(idx, weight)


def kernel(token_ids, weight):
    b, h = token_ids.shape
    t = token_ids.T.astype(jnp.int32)
    r = t.reshape(_H, _BT, _CHUNK).transpose(1, 0, 2)
    idx = (r.reshape(_NW, _BT_PER_W, _H, _CHUNK)
             .transpose(0, 2, 1, 3)
             .reshape(_NW, _N_CHUNKS, _CHUNK))
    out4 = _emb_call(idx, weight)
    out5 = out4.reshape(_H, 8, _BT, 8, _CHUNK)
    return out5.transpose(2, 4, 0, 1, 3).reshape(b, h, _D)


# speedup vs baseline: 1.4855x; 1.4855x over previous
import functools

import jax
import jax.numpy as jnp
from jax import lax
from jax.experimental import pallas as pl
from jax.experimental.pallas import tpu as pltpu
from jax.experimental.pallas import tpu_sc as plsc

_D = 64
_CHUNK = 128
_H = 50
_BT = 128

_INFO = plsc.get_sparse_core_info()
_NC = _INFO.num_cores
_NS = _INFO.num_subcores
_NW = _NC * _NS
_BT_PER_W = _BT // _NW
_N_CHUNKS = _H * _BT_PER_W


def _emb_body(idx_hbm, table_hbm, out_hbm, idx_v, rows0, rows1, patch0,
              patch1, gsem, psem):
    wid = lax.axis_index("s") * _NC + lax.axis_index("c")
    pltpu.sync_copy(
        idx_hbm.at[:, pl.ds(wid * (_BT_PER_W * _CHUNK), _BT_PER_W * _CHUNK)],
        idx_v)

    rows = (rows0, rows1)
    patch = (patch0, patch1)

    iota = lax.iota(jnp.int32, 16)
    rot = [(iota + s) % 16 for s in range(16)]
    srot = [rot[s] * 128 + iota for s in range(16)]

    def idx_slice(j):
        h = j // _BT_PER_W
        k = j % _BT_PER_W
        return idx_v.at[h, pl.ds(_CHUNK * k, _CHUNK)]

    def issue_gather(j, b):
        pltpu.async_copy(table_hbm.at[idx_slice(j)], rows[b], gsem.at[b])

    def wait_gather(j, b):
        pltpu.make_async_copy(
            table_hbm.at[idx_slice(j)], rows[b], gsem.at[b]).wait()

    def transpose_chunk(b):
        def tb_body(tb, carry):
            bl0 = 16 * tb
            bl_vec = iota + bl0
            for td in range(_D // 16):
                base = 16 * td * 128 + bl0
                for s in range(16):
                    vals = plsc.load_gather(rows[b], [bl_vec, rot[s] + 16 * td])
                    plsc.store_scatter(patch[b], [srot[s] + base], vals)
            return carry

        lax.fori_loop(0, _CHUNK // 16, tb_body, 0)

    def out_tile(j, dt):
        h = j // _BT_PER_W
        bt = wid * _BT_PER_W + (j % _BT_PER_W)
        return out_hbm.at[h, dt, bt]

    def issue_writes(j, b):
        for dt in range(8):
            pltpu.async_copy(
                patch[b].at[pl.ds(1024 * dt, 1024)], out_tile(j, dt),
                psem.at[b])

    def wait_writes(j, b):
        for dt in range(8):
            pltpu.make_async_copy(
                patch[b].at[pl.ds(1024 * dt, 1024)], out_tile(j, dt),
                psem.at[b]).wait()

    issue_gather(0, 0)

    def pair(p, carry):
        for s in range(2):
            j = 2 * p + s
            wait_gather(j, s)

            @pl.when(j + 1 < _N_CHUNKS)
            def _():
                issue_gather(j + 1, 1 - s)

            @pl.when(j >= 2)
            def _():
                wait_writes(j - 2, s)

            transpose_chunk(s)
            issue_writes(j, s)
        return carry

    lax.fori_loop(0, _N_CHUNKS // 2, pair, 0)

    wait_writes(_N_CHUNKS - 2, 0)
    wait_writes(_N_CHUNKS - 1, 1)


@jax.jit
def _emb_call(idx, weight):
    mesh = plsc.VectorSubcoreMesh(core_axis_name="c", subcore_axis_name="s")
    run = pl.kernel(
        _emb_body,
        out_type=jax.ShapeDtypeStruct((_H, 8, _BT, 8 * _CHUNK), jnp.float32),
        mesh=mesh,
        scratch_types=[
            pltpu.VMEM((_H, _BT_PER_W * _CHUNK), jnp.int32),
            pltpu.VMEM((_CHUNK, _D), jnp.float32),
            pltpu.VMEM((_CHUNK, _D), jnp.float32),
            pltpu.VMEM((_D * _CHUNK,), jnp.float32),
            pltpu.VMEM((_D * _CHUNK,), jnp.float32),
            pltpu.SemaphoreType.DMA((2,)),
            pltpu.SemaphoreType.DMA((2,)),
        ],
        compiler_params=pltpu.CompilerParams(
            use_tc_tiling_on_sc=False, needs_layout_passes=False),
    )
    return run(idx, weight)


def kernel(token_ids, weight):
    b, h = token_ids.shape
    idx = token_ids.T.astype(jnp.int32)
    out4 = _emb_call(idx, weight)
    out5 = out4.reshape(_H, 8, _BT, 8, _CHUNK)
    return out5.transpose(2, 4, 0, 1, 3).reshape(b, h, _D)
